# baseline (speedup 1.0000x reference)
import functools

import jax
import jax.numpy as jnp
from jax.experimental import pallas as pl
from jax.experimental.pallas import tpu as pltpu

_G = 16
_CW = 64
_R = 2000
_NEG = -jnp.inf


def _dense_kernel(mean_ref, xin_ref, Wl_ref, bl_ref, Wr_ref, wn_ref,
                  h_ref, s_ref):
    h = mean_ref[...] @ Wl_ref[...] + bl_ref[...] + xin_ref[...] @ Wr_ref[...]
    h = jax.nn.relu(h)
    h_ref[...] = h
    s_ref[...] = jnp.tanh(h @ wn_ref[...])


def _dense(mean, xin, Wl, bl, Wr, wn, N):
    F = xin.shape[1]
    grid = N // _R
    return pl.pallas_call(
        _dense_kernel,
        grid=(grid,),
        in_specs=[
            pl.BlockSpec((_R, F), lambda i: (i, 0)),
            pl.BlockSpec((_R, F), lambda i: (i, 0)),
            pl.BlockSpec((F, _CW), lambda i: (0, 0)),
            pl.BlockSpec((1, _CW), lambda i: (0, 0)),
            pl.BlockSpec((F, _CW), lambda i: (0, 0)),
            pl.BlockSpec((_CW, 1), lambda i: (0, 0)),
        ],
        out_specs=[
            pl.BlockSpec((_R, _CW), lambda i: (i, 0)),
            pl.BlockSpec((_R, 1), lambda i: (i, 0)),
        ],
        out_shape=[
            jax.ShapeDtypeStruct((N, _CW), jnp.float32),
            jax.ShapeDtypeStruct((N, 1), jnp.float32),
        ],
    )(mean, xin, Wl, bl.reshape(1, _CW), Wr, wn)


def _scale_kernel(h_ref, s_ref, k_ref, o_ref):
    o_ref[...] = h_ref[...] * s_ref[...] * k_ref[...]


def _scale(h, score, keepf, N):
    grid = N // _R
    return pl.pallas_call(
        _scale_kernel,
        grid=(grid,),
        in_specs=[
            pl.BlockSpec((_R, _CW), lambda i: (i, 0)),
            pl.BlockSpec((_R, 1), lambda i: (i, 0)),
            pl.BlockSpec((_R, 1), lambda i: (i, 0)),
        ],
        out_specs=pl.BlockSpec((_R, _CW), lambda i: (i, 0)),
        out_shape=jax.ShapeDtypeStruct((N, _CW), jnp.float32),
    )(h, score, keepf)


def _pools_mlp_kernel(h1_ref, k1_ref, h2_ref, k2_ref, b_ref,
                      W1_ref, b1_ref, W2_ref, b2_ref, W3_ref, b3_ref,
                      o_ref, s1, c1, m1, s2, c2, m2):
    i = pl.program_id(0)
    n = pl.num_programs(0)

    @pl.when(i == 0)
    def _init():
        s1[...] = jnp.zeros_like(s1)
        s2[...] = jnp.zeros_like(s2)
        c1[...] = jnp.zeros_like(c1)
        c2[...] = jnp.zeros_like(c2)
        m1[...] = jnp.full(m1.shape, _NEG, jnp.float32)
        m2[...] = jnp.full(m2.shape, _NEG, jnp.float32)

    bb = jnp.broadcast_to(b_ref[...], (_R, _G))
    gids = jax.lax.broadcasted_iota(jnp.int32, (_R, _G), 1)
    ohf = (bb == gids).astype(jnp.float32)
    ones8 = jnp.ones((_R, 8), jnp.float32)
    cdims = (((0,), (0,)), ((), ()))
    for (h_ref, k_ref, s, c, m) in ((h1_ref, k1_ref, s1, c1, m1),
                                    (h2_ref, k2_ref, s2, c2, m2)):
        hk = h_ref[...]
        ohk = ohf * jnp.broadcast_to(k_ref[...], (_R, _G))
        s[...] += jax.lax.dot_general(ohk, hk, cdims)
        c[...] += jax.lax.dot_general(ohk, ones8, cdims)
        for g in range(_G):
            m64 = jnp.broadcast_to(ohk[:, g:g + 1], (_R, _CW))
            sel = jnp.where(m64 > 0.5, hk, _NEG)
            m[g:g + 1, :] = jnp.maximum(m[g:g + 1, :],
                                        jnp.max(sel, axis=0, keepdims=True))

    @pl.when(i == n - 1)
    def _fin():
        gap1 = s1[...] / jnp.clip(c1[:, 0:1], 1.0, None)
        gap2 = s2[...] / jnp.clip(c2[:, 0:1], 1.0, None)
        z = jnp.concatenate([m1[...] + m2[...], gap1 + gap2], axis=1)
        a = jax.nn.relu(z @ W1_ref[...] + b1_ref[...])
        a = jax.nn.relu(a @ W2_ref[...] + b2_ref[...])
        o_ref[...] = a @ W3_ref[...] + b3_ref[...]


def _pools_mlp(h1k, keep1f, h2k, keep2f, batch2d, W1, b1, W2, b2, W3, b3, N):
    grid = N // _R
    col = lambda w: pl.BlockSpec(w.shape, lambda i: (0, 0))
    return pl.pallas_call(
        _pools_mlp_kernel,
        grid=(grid,),
        in_specs=[
            pl.BlockSpec((_R, _CW), lambda i: (i, 0)),
            pl.BlockSpec((_R, 1), lambda i: (i, 0)),
            pl.BlockSpec((_R, _CW), lambda i: (i, 0)),
            pl.BlockSpec((_R, 1), lambda i: (i, 0)),
            pl.BlockSpec((_R, 1), lambda i: (i, 0)),
            col(W1), pl.BlockSpec((1, 128), lambda i: (0, 0)),
            col(W2), pl.BlockSpec((1, _CW), lambda i: (0, 0)),
            col(W3), pl.BlockSpec((1, 1), lambda i: (0, 0)),
        ],
        out_specs=pl.BlockSpec((_G, 1), lambda i: (0, 0)),
        out_shape=jax.ShapeDtypeStruct((_G, 1), jnp.float32),
        scratch_shapes=[
            pltpu.VMEM((_G, _CW), jnp.float32),
            pltpu.VMEM((_G, 8), jnp.float32),
            pltpu.VMEM((_G, _CW), jnp.float32),
            pltpu.VMEM((_G, _CW), jnp.float32),
            pltpu.VMEM((_G, 8), jnp.float32),
            pltpu.VMEM((_G, _CW), jnp.float32),
        ],
    )(h1k, keep1f, h2k, keep2f, batch2d,
      W1, b1.reshape(1, 128), W2, b2.reshape(1, _CW), W3, b3.reshape(1, 1))


def _keep_mask(score, batch, N):
    order = jnp.lexsort((-score, batch))
    b_s = batch[order]
    cnt = jax.ops.segment_sum(jnp.ones((N,), jnp.int32), batch,
                              num_segments=_G)
    k = (cnt + 1) // 2
    start = jnp.cumsum(cnt) - cnt
    g = jnp.clip(b_s, 0, _G - 1)
    rank = jnp.arange(N, dtype=jnp.int32) - start[g]
    keep_s = (b_s < _G) & (rank < k[g])
    return jnp.zeros((N,), jnp.bool_).at[order].set(keep_s)


def kernel(x, edge_index, batch, Wl1, bl1, Wr1, Wl2, bl2, Wr2, Wl3, bl3, Wr3,
           w1, w2, w3, w4, Wg, bg, W1, b1, W2, b2, W3, b3):
    N = x.shape[0]
    src, dst = edge_index[0], edge_index[1]

    agg1 = jax.ops.segment_sum(x[src], dst, num_segments=N)
    cnt1 = jax.ops.segment_sum(jnp.ones(src.shape, jnp.float32), dst,
                               num_segments=N)
    mean1 = agg1 / jnp.clip(cnt1, 1.0, None)[:, None]
    pad = lambda a: jnp.pad(a, ((0, 0), (0, 8 - a.shape[1])))
    wn1 = (w1 / jnp.linalg.norm(w1)).reshape(_CW, 1)
    h1, score1 = _dense(pad(mean1), pad(x), pad(Wl1.T).T, bl1, pad(Wr1.T).T,
                        wn1, N)

    keep1 = _keep_mask(score1[:, 0], batch, N)
    keep1f = keep1.astype(jnp.float32)[:, None]
    h1k = _scale(h1, score1, keep1f, N)

    live = keep1[src] & keep1[dst]
    dst2 = jnp.where(live, dst, N)
    agg2 = jax.ops.segment_sum(h1k[src] * live[:, None].astype(jnp.float32),
                               dst2, num_segments=N)
    cnt2 = jax.ops.segment_sum(live.astype(jnp.float32), dst2, num_segments=N)
    mean2 = agg2 / jnp.clip(cnt2, 1.0, None)[:, None]
    wn2 = (w2 / jnp.linalg.norm(w2)).reshape(_CW, 1)
    h2, score2 = _dense(mean2, h1k, Wl2, bl2, Wr2, wn2, N)

    batch_k1 = jnp.where(keep1, batch, _G)
    keep2 = _keep_mask(score2[:, 0], batch_k1, N)
    keep2f = keep2.astype(jnp.float32)[:, None]
    h2k = _scale(h2, score2, keep2f, N)

    return _pools_mlp(h1k, keep1f, h2k, keep2f, batch[:, None],
                      W1, b1, W2, b2, W3, b3, N)
